# Initial kernel scaffold; baseline (speedup 1.0000x reference)
import jax
import jax.numpy as jnp
from jax.experimental import pallas as pl


def kernel(x, graph_list, mapping_list, W_cheb, b_cheb, fc1_w, fc1_b):
    raise NotImplementedError("write your pallas kernel here")


# speedup vs baseline: 3.3928x; 3.3928x over previous
import functools
import jax
import jax.numpy as jnp
from jax import lax
from jax.experimental import pallas as pl
from jax.experimental.pallas import tpu as pltpu
from jax.experimental.pallas import tpu_sc as plsc

B, N, H, E, K, G, C = 8, 10000, 15, 160000, 25, 64, 6
HP = 16
F = B * HP
NP = 10240
NW = 32
NSUB = 16
ROWS_PER_TILE = NP // NW
ROWS_PER_SUB = NP // NSUB
CH = 128
NCH = 40
EPAD = NW * NCH * CH
DEGW = 16

@functools.lru_cache(maxsize=1)
def _sc_kernels():
    mesh = plsc.VectorSubcoreMesh(core_axis_name="c", subcore_axis_name="s")

    @functools.partial(
        pl.kernel,
        out_type=jax.ShapeDtypeStruct((2, NP, F), jnp.float32),
        mesh=mesh,
        scratch_types=[
            pltpu.VMEM((NCH, CH), jnp.int32),
            pltpu.VMEM((NCH, CH), jnp.int32),
            pltpu.VMEM((CH, F), jnp.float32),
            pltpu.VMEM((CH, F), jnp.float32),
            pltpu.VMEM_SHARED((NP, F), jnp.float32),
        ],
    )
    def spmm_kernel(y_hbm, colidx_hbm, rowidx_hbm, acc_out,
                    cidx_v, ridx_v, gbuf, zb, acc_sh):
        c = lax.axis_index("c")
        s = lax.axis_index("s")
        wid = s * 2 + c

        @pl.loop(0, CH)
        def _fill(i):
            for j in range(F // 16):
                zb[i, pl.ds(j * 16, 16)] = jnp.zeros((16,), jnp.float32)

        @pl.loop(0, ROWS_PER_SUB // CH)
        def _clear(t):
            pltpu.sync_copy(zb, acc_sh.at[pl.ds(s * ROWS_PER_SUB + t * CH, CH)])

        plsc.subcore_barrier()

        pltpu.sync_copy(colidx_hbm.at[wid], cidx_v)
        pltpu.sync_copy(rowidx_hbm.at[wid], ridx_v)

        @pl.loop(0, NCH)
        def _edges(j):
            pltpu.sync_copy(y_hbm.at[cidx_v.at[j]], gbuf)
            pltpu.sync_copy(gbuf, acc_sh.at[ridx_v.at[j]], add=True)

        plsc.subcore_barrier()

        pltpu.sync_copy(
            acc_sh.at[pl.ds(s * ROWS_PER_SUB, ROWS_PER_SUB)],
            acc_out.at[c].at[pl.ds(s * ROWS_PER_SUB, ROWS_PER_SUB)])

    return spmm_kernel


_NBLK_TC = 8
_RB = NP // _NBLK_TC


def _setup_body(z0_ref, deg2_ref, dis_ref, y0_ref):
    deg = deg2_ref[0, :, 0:1] + deg2_ref[1, :, 0:1]
    dis = jnp.where(deg > 0.0, lax.rsqrt(jnp.maximum(deg, 1e-30)), 0.0)
    disrep = jnp.broadcast_to(dis, (_RB, F))
    dis_ref[...] = disrep
    y0_ref[...] = disrep * z0_ref[...]


def _setup_call(z0, deg2):
    return pl.pallas_call(
        _setup_body,
        grid=(_NBLK_TC,),
        in_specs=[
            pl.BlockSpec((_RB, F), lambda i: (i, 0)),
            pl.BlockSpec((2, _RB, F), lambda i: (0, i, 0)),
        ],
        out_specs=[
            pl.BlockSpec((_RB, F), lambda i: (i, 0)),
            pl.BlockSpec((_RB, F), lambda i: (i, 0)),
        ],
        out_shape=[
            jax.ShapeDtypeStruct((NP, F), jnp.float32),
            jax.ShapeDtypeStruct((NP, F), jnp.float32),
        ],
    )(z0, deg2)


def _combine_body(first, acc2_ref, prev_ref, dis_ref, tx_ref, y_ref):
    dis = dis_ref[...]
    acc = acc2_ref[0] + acc2_ref[1]
    if first:
        tx = -dis * acc
    else:
        tx = -2.0 * dis * acc - prev_ref[...]
    tx_ref[...] = tx
    y_ref[...] = dis * tx


def _combine_call(first, acc2, prev, dis):
    return pl.pallas_call(
        functools.partial(_combine_body, first),
        grid=(_NBLK_TC,),
        in_specs=[pl.BlockSpec((2, _RB, F), lambda i: (0, i, 0))] +
                 [pl.BlockSpec((_RB, F), lambda i: (i, 0))] * 2,
        out_specs=[pl.BlockSpec((_RB, F), lambda i: (i, 0))] * 2,
        out_shape=[
            jax.ShapeDtypeStruct((NP, F), jnp.float32),
            jax.ShapeDtypeStruct((NP, F), jnp.float32),
        ],
    )(acc2, prev, dis)


_FNB = 10
_RBF = NP // _FNB


def _final_body(*refs):
    tx_refs = refs[:K]
    wbig_ref, w2n_ref, gmask_ref, sel8_ref, sel6_ref, bvec_ref, fbias_ref = \
        refs[K:K + 7]
    out_ref = refs[K + 7]
    m_acc = refs[K + 8]

    i = pl.program_id(0)

    h = jnp.zeros((_RBF, 512), jnp.float32)
    for k in range(K):
        h = h + lax.dot(tx_refs[k][...], wbig_ref[k],
                        preferred_element_type=jnp.float32)
    h = jnp.maximum(h + bvec_ref[...], 0.0)

    m = lax.dot_general(h, w2n_ref[...], (((0,), (0,)), ((), ())),
                        preferred_element_type=jnp.float32)

    @pl.when(i == 0)
    def _():
        m_acc[...] = jnp.zeros_like(m_acc)

    m_acc[...] += m

    @pl.when(i == _FNB - 1)
    def _():
        md = m_acc[...] * gmask_ref[...]
        t = lax.dot(sel8_ref[...], md, preferred_element_type=jnp.float32)
        logits = lax.dot(t, sel6_ref[...],
                         preferred_element_type=jnp.float32) + fbias_ref[...]
        mx = jnp.max(logits, axis=1, keepdims=True)
        sh = logits - mx
        lse = jnp.log(jnp.sum(jnp.exp(sh), axis=1, keepdims=True))
        out_ref[...] = sh - lse


def _final_call(tx_list, wbig, w2n, gmask, sel8, sel6, bvec, fbias):
    blk = pl.BlockSpec((_RBF, F), lambda i: (i, 0))
    in_specs = [blk] * K + [
        pl.BlockSpec((K, F, 512), lambda i: (0, 0, 0)),
        pl.BlockSpec((_RBF, 384), lambda i: (i, 0)),
        pl.BlockSpec((512, 384), lambda i: (0, 0)),
        pl.BlockSpec((8, 512), lambda i: (0, 0)),
        pl.BlockSpec((384, 6), lambda i: (0, 0)),
        pl.BlockSpec((1, 512), lambda i: (0, 0)),
        pl.BlockSpec((1, 6), lambda i: (0, 0)),
    ]
    return pl.pallas_call(
        _final_body,
        grid=(_FNB,),
        in_specs=in_specs,
        out_specs=pl.BlockSpec((8, 6), lambda i: (0, 0)),
        out_shape=jax.ShapeDtypeStruct((8, 6), jnp.float32),
        scratch_shapes=[pltpu.VMEM((512, 384), jnp.float32)],
    )(*tx_list, wbig, w2n, gmask, sel8, sel6, bvec, fbias)


def kernel(x, graph_list, mapping_list, W_cheb, b_cheb, fc1_w, fc1_b):
    f32 = jnp.float32

    xp = jnp.pad(x, ((0, 0), (0, 0), (0, HP - H)))
    z0 = jnp.transpose(xp, (1, 0, 2)).reshape(N, F)
    z0 = jnp.pad(z0, ((0, NP - N), (0, 0)))

    row = graph_list[0, 0].astype(jnp.int32)
    col = graph_list[0, 1].astype(jnp.int32)
    pad_e = EPAD - E
    rowp = jnp.concatenate([row, jnp.full((pad_e,), NP - 1, jnp.int32)])
    colp = jnp.concatenate([col, jnp.full((pad_e,), NP - 1, jnp.int32)])
    rowidx = rowp.reshape(NW, NCH, CH)
    colidx = colp.reshape(NW, NCH, CH)

    spmm_kernel = _sc_kernels()
    ones_tab = jnp.ones((NP, F), f32)
    deg2 = spmm_kernel(ones_tab, rowidx, rowidx)
    dis, y = _setup_call(z0, deg2)

    tx_list = [z0]
    prev2 = z0
    for k in range(1, K):
        acc2 = spmm_kernel(y, colidx, rowidx)
        tx, y_next = _combine_call(k == 1, acc2, prev2, dis)
        prev2 = tx_list[-1]
        tx_list.append(tx)
        y = y_next

    Wp = jnp.pad(W_cheb.astype(f32), ((0, 0), (0, HP - H), (0, 0)))
    eye8 = jnp.eye(8, dtype=f32)
    wbig = jax.vmap(lambda w: jnp.kron(eye8, w))(Wp)
    w2n = fc1_w.astype(f32).reshape(N, G, C).transpose(0, 2, 1).reshape(N, C * G)
    w2n = jnp.pad(w2n, ((0, NP - N), (0, 0)))
    pidx = jnp.arange(512) % 64
    qidx = jnp.arange(384) % 64
    gmask = (pidx[:, None] == qidx[None, :]).astype(f32)
    sel8 = (jnp.arange(512)[None, :] // 64 ==
            jnp.arange(8)[:, None]).astype(f32)
    sel6 = (jnp.arange(384)[:, None] // 64 ==
            jnp.arange(6)[None, :]).astype(f32)
    bvec = jnp.tile(b_cheb.astype(f32), 8)[None, :]
    fbias = fc1_b.astype(f32)[None, :]

    return _final_call(tx_list, wbig, w2n, gmask, sel8, sel6, bvec, fbias)


# speedup vs baseline: 3.6190x; 1.0667x over previous
import functools
import jax
import jax.numpy as jnp
from jax import lax
from jax.experimental import pallas as pl
from jax.experimental.pallas import tpu as pltpu
from jax.experimental.pallas import tpu_sc as plsc

B, N, H, E, K, G, C = 8, 10000, 15, 160000, 25, 64, 6
HP = 16
F = B * HP
NP = 10240
NW = 32
NSUB = 16
ROWS_PER_TILE = NP // NW
ROWS_PER_SUB = NP // NSUB
CH = 128
NCH = 40
EPAD = NW * NCH * CH
DEGW = 16

@functools.lru_cache(maxsize=1)
def _sc_kernels():
    mesh = plsc.VectorSubcoreMesh(core_axis_name="c", subcore_axis_name="s")

    @functools.partial(
        pl.kernel,
        out_type=jax.ShapeDtypeStruct((2, NP, F), jnp.float32),
        mesh=mesh,
        scratch_types=[
            pltpu.VMEM((NCH, CH), jnp.int32),
            pltpu.VMEM((NCH, CH), jnp.int32),
            pltpu.VMEM((CH, F), jnp.float32),
            pltpu.VMEM((CH, F), jnp.float32),
            pltpu.VMEM_SHARED((NP, F), jnp.float32),
            pltpu.SemaphoreType.DMA,
        ],
    )
    def spmm_kernel(y_hbm, colidx_hbm, rowidx_hbm, acc_out,
                    cidx_v, ridx_v, gbuf, zb, acc_sh, gsem):
        c = lax.axis_index("c")
        s = lax.axis_index("s")
        wid = s * 2 + c

        @pl.loop(0, CH)
        def _fill(i):
            for j in range(F // 16):
                zb[i, pl.ds(j * 16, 16)] = jnp.zeros((16,), jnp.float32)

        pltpu.sync_copy(colidx_hbm.at[wid], cidx_v)
        pltpu.sync_copy(rowidx_hbm.at[wid], ridx_v)

        @pl.loop(0, ROWS_PER_SUB // CH)
        def _clear(t):
            pltpu.sync_copy(zb, acc_sh.at[pl.ds(s * ROWS_PER_SUB + t * CH, CH)])

        plsc.subcore_barrier()

        gbufs = [gbuf, zb]

        def g_start(j, b):
            pltpu.async_copy(y_hbm.at[cidx_v.at[j]], gbufs[b], gsem)

        def g_wait(b):
            pltpu.make_async_copy(y_hbm.at[pl.ds(0, CH)], gbufs[b],
                                  gsem).wait()

        def step(j, bcur, issue_next):
            g_wait(bcur)
            if issue_next:
                g_start(j + 1, 1 - bcur)
            pltpu.sync_copy(gbufs[bcur], acc_sh.at[ridx_v.at[j]], add=True)

        g_start(0, 0)

        @pl.loop(0, NCH // 2 - 1)
        def _edges(r):
            step(2 * r, 0, True)
            step(2 * r + 1, 1, True)

        step(NCH - 2, 0, True)
        step(NCH - 1, 1, False)

        plsc.subcore_barrier()

        pltpu.sync_copy(
            acc_sh.at[pl.ds(s * ROWS_PER_SUB, ROWS_PER_SUB)],
            acc_out.at[c].at[pl.ds(s * ROWS_PER_SUB, ROWS_PER_SUB)])

    return spmm_kernel


_NBLK_TC = 8
_RB = NP // _NBLK_TC


def _setup_body(z0_ref, deg2_ref, dis_ref, y0_ref):
    deg = deg2_ref[0, :, 0:1] + deg2_ref[1, :, 0:1]
    dis = jnp.where(deg > 0.0, lax.rsqrt(jnp.maximum(deg, 1e-30)), 0.0)
    disrep = jnp.broadcast_to(dis, (_RB, F))
    dis_ref[...] = disrep
    y0_ref[...] = disrep * z0_ref[...]


def _setup_call(z0, deg2):
    return pl.pallas_call(
        _setup_body,
        grid=(_NBLK_TC,),
        in_specs=[
            pl.BlockSpec((_RB, F), lambda i: (i, 0)),
            pl.BlockSpec((2, _RB, F), lambda i: (0, i, 0)),
        ],
        out_specs=[
            pl.BlockSpec((_RB, F), lambda i: (i, 0)),
            pl.BlockSpec((_RB, F), lambda i: (i, 0)),
        ],
        out_shape=[
            jax.ShapeDtypeStruct((NP, F), jnp.float32),
            jax.ShapeDtypeStruct((NP, F), jnp.float32),
        ],
    )(z0, deg2)


def _combine_body(first, acc2_ref, prev_ref, dis_ref, tx_ref, y_ref):
    dis = dis_ref[...]
    acc = acc2_ref[0] + acc2_ref[1]
    if first:
        tx = -dis * acc
    else:
        tx = -2.0 * dis * acc - prev_ref[...]
    tx_ref[...] = tx
    y_ref[...] = dis * tx


def _combine_call(first, acc2, prev, dis):
    return pl.pallas_call(
        functools.partial(_combine_body, first),
        grid=(_NBLK_TC,),
        in_specs=[pl.BlockSpec((2, _RB, F), lambda i: (0, i, 0))] +
                 [pl.BlockSpec((_RB, F), lambda i: (i, 0))] * 2,
        out_specs=[pl.BlockSpec((_RB, F), lambda i: (i, 0))] * 2,
        out_shape=[
            jax.ShapeDtypeStruct((NP, F), jnp.float32),
            jax.ShapeDtypeStruct((NP, F), jnp.float32),
        ],
    )(acc2, prev, dis)


_FNB = 10
_RBF = NP // _FNB


def _final_body(*refs):
    tx_refs = refs[:K]
    wbig_ref, w2n_ref, gmask_ref, sel8_ref, sel6_ref, bvec_ref, fbias_ref = \
        refs[K:K + 7]
    out_ref = refs[K + 7]
    m_acc = refs[K + 8]

    i = pl.program_id(0)

    h = jnp.zeros((_RBF, 512), jnp.float32)
    for k in range(K):
        h = h + lax.dot(tx_refs[k][...], wbig_ref[k],
                        preferred_element_type=jnp.float32)
    h = jnp.maximum(h + bvec_ref[...], 0.0)

    m = lax.dot_general(h, w2n_ref[...], (((0,), (0,)), ((), ())),
                        preferred_element_type=jnp.float32)

    @pl.when(i == 0)
    def _():
        m_acc[...] = jnp.zeros_like(m_acc)

    m_acc[...] += m

    @pl.when(i == _FNB - 1)
    def _():
        md = m_acc[...] * gmask_ref[...]
        t = lax.dot(sel8_ref[...], md, preferred_element_type=jnp.float32)
        logits = lax.dot(t, sel6_ref[...],
                         preferred_element_type=jnp.float32) + fbias_ref[...]
        mx = jnp.max(logits, axis=1, keepdims=True)
        sh = logits - mx
        lse = jnp.log(jnp.sum(jnp.exp(sh), axis=1, keepdims=True))
        out_ref[...] = sh - lse


def _final_call(tx_list, wbig, w2n, gmask, sel8, sel6, bvec, fbias):
    blk = pl.BlockSpec((_RBF, F), lambda i: (i, 0))
    in_specs = [blk] * K + [
        pl.BlockSpec((K, F, 512), lambda i: (0, 0, 0)),
        pl.BlockSpec((_RBF, 384), lambda i: (i, 0)),
        pl.BlockSpec((512, 384), lambda i: (0, 0)),
        pl.BlockSpec((8, 512), lambda i: (0, 0)),
        pl.BlockSpec((384, 6), lambda i: (0, 0)),
        pl.BlockSpec((1, 512), lambda i: (0, 0)),
        pl.BlockSpec((1, 6), lambda i: (0, 0)),
    ]
    return pl.pallas_call(
        _final_body,
        grid=(_FNB,),
        in_specs=in_specs,
        out_specs=pl.BlockSpec((8, 6), lambda i: (0, 0)),
        out_shape=jax.ShapeDtypeStruct((8, 6), jnp.float32),
        scratch_shapes=[pltpu.VMEM((512, 384), jnp.float32)],
    )(*tx_list, wbig, w2n, gmask, sel8, sel6, bvec, fbias)


def kernel(x, graph_list, mapping_list, W_cheb, b_cheb, fc1_w, fc1_b):
    f32 = jnp.float32

    xp = jnp.pad(x, ((0, 0), (0, 0), (0, HP - H)))
    z0 = jnp.transpose(xp, (1, 0, 2)).reshape(N, F)
    z0 = jnp.pad(z0, ((0, NP - N), (0, 0)))

    row = graph_list[0, 0].astype(jnp.int32)
    col = graph_list[0, 1].astype(jnp.int32)
    pad_e = EPAD - E
    rowp = jnp.concatenate([row, jnp.full((pad_e,), NP - 1, jnp.int32)])
    colp = jnp.concatenate([col, jnp.full((pad_e,), NP - 1, jnp.int32)])
    rowidx = rowp.reshape(NW, NCH, CH)
    colidx = colp.reshape(NW, NCH, CH)

    spmm_kernel = _sc_kernels()
    ones_tab = jnp.ones((NP, F), f32)
    deg2 = spmm_kernel(ones_tab, rowidx, rowidx)
    dis, y = _setup_call(z0, deg2)

    tx_list = [z0]
    prev2 = z0
    for k in range(1, K):
        acc2 = spmm_kernel(y, colidx, rowidx)
        tx, y_next = _combine_call(k == 1, acc2, prev2, dis)
        prev2 = tx_list[-1]
        tx_list.append(tx)
        y = y_next

    Wp = jnp.pad(W_cheb.astype(f32), ((0, 0), (0, HP - H), (0, 0)))
    eye8 = jnp.eye(8, dtype=f32)
    wbig = jax.vmap(lambda w: jnp.kron(eye8, w))(Wp)
    w2n = fc1_w.astype(f32).reshape(N, G, C).transpose(0, 2, 1).reshape(N, C * G)
    w2n = jnp.pad(w2n, ((0, NP - N), (0, 0)))
    pidx = jnp.arange(512) % 64
    qidx = jnp.arange(384) % 64
    gmask = (pidx[:, None] == qidx[None, :]).astype(f32)
    sel8 = (jnp.arange(512)[None, :] // 64 ==
            jnp.arange(8)[:, None]).astype(f32)
    sel6 = (jnp.arange(384)[:, None] // 64 ==
            jnp.arange(6)[None, :]).astype(f32)
    bvec = jnp.tile(b_cheb.astype(f32), 8)[None, :]
    fbias = fc1_b.astype(f32)[None, :]

    return _final_call(tx_list, wbig, w2n, gmask, sel8, sel6, bvec, fbias)
